# baseline (speedup 1.0000x reference)
import functools

import jax
import jax.numpy as jnp
from jax import lax
from jax.experimental import pallas as pl
from jax.experimental.pallas import tpu as pltpu
from jax.experimental.pallas import tpu_sc as plsc

NC = 2
NS = 16
NW = NC * NS
L = 16

B = 16384
EMB = 64
HIST = 200
BPW = B // NW
NCHUNK = BPW // 128


def _body(gidx_hbm, aidx_hbm, sidx_hbm, song_hbm, genre_hbm, artist_hbm,
          wb_hbm, out_hbm,
          sidx_v, cidx_v, rows_v, grows_v, arows_v, wv, outv, sem_s, sem_c):
    c = lax.axis_index("c")
    s = lax.axis_index("s")
    wid = s * NC + c
    base = wid * BPW

    for j in range(NCHUNK):
        pltpu.sync_copy(sidx_hbm.at[pl.ds(base + j * 128, 128)], sidx_v.at[j])
    song_cps = [
        pltpu.async_copy(song_hbm.at[sidx_v.at[j]],
                         rows_v.at[pl.ds(j * 128, 128)], sem_s)
        for j in range(NCHUNK)
    ]

    pltpu.sync_copy(gidx_hbm.at[pl.ds(0, 128)], cidx_v.at[0])
    pltpu.sync_copy(gidx_hbm.at[pl.ds(128, 72)], cidx_v.at[1, pl.ds(0, 72)])
    pltpu.sync_copy(aidx_hbm.at[pl.ds(0, 128)], cidx_v.at[2])
    pltpu.sync_copy(aidx_hbm.at[pl.ds(128, 72)], cidx_v.at[3, pl.ds(0, 72)])
    const_cps = [
        pltpu.async_copy(genre_hbm.at[cidx_v.at[0]],
                         grows_v.at[pl.ds(0, 128)], sem_c),
        pltpu.async_copy(genre_hbm.at[cidx_v.at[1, pl.ds(0, 72)]],
                         grows_v.at[pl.ds(128, 72)], sem_c),
        pltpu.async_copy(artist_hbm.at[cidx_v.at[2]],
                         arows_v.at[pl.ds(0, 128)], sem_c),
        pltpu.async_copy(artist_hbm.at[cidx_v.at[3, pl.ds(0, 72)]],
                         arows_v.at[pl.ds(128, 72)], sem_c),
    ]

    pltpu.sync_copy(wb_hbm, wv)

    for cp in const_cps:
        cp.wait()

    zeros = jnp.zeros((L,), jnp.float32)
    lane = lax.iota(jnp.int32, L)

    dnums = lax.GatherDimensionNumbers(
        offset_dims=(), collapsed_slice_dims=(0,), start_index_map=(0,))

    def lperm(v, idx):
        return lax.gather(v, idx[:, None], dnums, slice_sizes=(1,),
                          mode=lax.GatherScatterMode.PROMISE_IN_BOUNDS)

    def allsum(v):
        for step in (1, 2, 4, 8):
            v = v + lperm(v, lane ^ step)
        return v

    def accum(rows_ref):
        def it(r, accs):
            return tuple(accs[k] + rows_ref[r, pl.ds(16 * k, 16)]
                         for k in range(4))
        return lax.fori_loop(0, HIST, it, (zeros,) * 4)

    gsum = accum(grows_v)
    asum = accum(arows_v)

    wg = [wv[0, pl.ds(16 * k, 16)] for k in range(4)]
    wa = [wv[0, pl.ds(64 + 16 * k, 16)] for k in range(4)]
    ws = [wv[0, pl.ds(128 + 16 * k, 16)] for k in range(4)]
    tg = gsum[0] * wg[0] + gsum[1] * wg[1] + gsum[2] * wg[2] + gsum[3] * wg[3]
    ta = asum[0] * wa[0] + asum[1] * wa[1] + asum[2] * wa[2] + asum[3] * wa[3]
    bias = allsum(wv[0, pl.ds(192, 16)])
    cconst = (allsum(tg) + allsum(ta)) * (1.0 / HIST) + bias

    for cp in song_cps:
        cp.wait()

    def group(g, _):
        acc = zeros
        for r in range(L):
            row = g * L + r
            v = (rows_v[row, pl.ds(0, 16)] * ws[0]
                 + rows_v[row, pl.ds(16, 16)] * ws[1]
                 + rows_v[row, pl.ds(32, 16)] * ws[2]
                 + rows_v[row, pl.ds(48, 16)] * ws[3])
            acc = jnp.where(lane == r, allsum(v), acc)
        outv[pl.ds(g * L, L)] = acc + cconst
        return 0

    lax.fori_loop(0, BPW // L, group, 0)

    pltpu.sync_copy(outv, out_hbm.at[pl.ds(base, BPW)])


@jax.jit
def _run(gidx, aidx, sidx, song_table, genre_table, artist_table, wb):
    mesh = plsc.VectorSubcoreMesh(core_axis_name="c", subcore_axis_name="s",
                                  num_cores=NC, num_subcores=NS)
    return pl.kernel(
        _body,
        out_type=jax.ShapeDtypeStruct((B,), jnp.float32),
        mesh=mesh,
        scratch_types=[
            pltpu.VMEM((NCHUNK, 128), jnp.int32),
            pltpu.VMEM((4, 128), jnp.int32),
            pltpu.VMEM((BPW, EMB), jnp.float32),
            pltpu.VMEM((HIST, EMB), jnp.float32),
            pltpu.VMEM((HIST, EMB), jnp.float32),
            pltpu.VMEM((1, 208), jnp.float32),
            pltpu.VMEM((BPW,), jnp.float32),
            pltpu.SemaphoreType.DMA,
            pltpu.SemaphoreType.DMA,
        ],
        compiler_params=pltpu.CompilerParams(use_tc_tiling_on_sc=False),
    )(gidx, aidx, sidx, song_table, genre_table, artist_table, wb)


def kernel(genre_indices, artist_indices, song_indices, song_table,
           genre_table, artist_table, fc_w, fc_b):
    wb = jnp.pad(jnp.concatenate([fc_w.reshape(-1), fc_b.reshape(-1)]),
                 (0, 15)).reshape(1, 208)
    return _run(genre_indices.astype(jnp.int32),
                artist_indices.astype(jnp.int32),
                song_indices.astype(jnp.int32),
                song_table, genre_table, artist_table, wb)
